# baseline (speedup 1.0000x reference)
import functools

import jax
import jax.numpy as jnp
from jax import lax
from jax.experimental import pallas as pl
from jax.experimental.pallas import tpu as pltpu
from jax.experimental.pallas import tpu_sc as plsc

N_NODES = 10000
N_EDGES = 320000
D_FEAT = 128

NUM_CORES = 2
NUM_SUBCORES = 16
NUM_WORKERS = NUM_CORES * NUM_SUBCORES
CHUNK = 128
NUM_CHUNKS = 80
EPW = NUM_CHUNKS * CHUNK
LAST_CHUNKS = (N_EDGES - (NUM_WORKERS - 1) * EPW) // CHUNK
N_ROWS_PAD = 10240
ROWS_PER_TILE = N_ROWS_PAD // NUM_SUBCORES
ZROWS = 64


def _sc_spmm(ei_hbm, val_hbm, emb_hbm, out_hbm,
             ds0, sr0, vl0, ds1, sr1, vl1, ds2, sr2, vl2, ds3, sr3, vl3,
             rows0, rows1, zbuf_v, acc_sh,
             semi0, semi1, semi2, semi3, semr0, semr1, semw0, semw1, semz):
    c = lax.axis_index("c")
    s = lax.axis_index("s")
    wid = c * NUM_SUBCORES + s
    ebase = wid * EPW
    nck = jnp.where(wid == NUM_WORKERS - 1, LAST_CHUNKS, NUM_CHUNKS)

    sets = ((ds0, sr0, vl0, semi0), (ds1, sr1, vl1, semi1),
            (ds2, sr2, vl2, semi2), (ds3, sr3, vl3, semi3))
    rbufs = ((rows0, semr0), (rows1, semr1))
    wsems = (semw0, semw1)

    def fire_idx(ci, k):
        dsb, srb, vlb, semi = sets[k]
        off = pl.multiple_of(ebase + ci * CHUNK, 8)
        pltpu.async_copy(ei_hbm.at[0, pl.ds(off, CHUNK)], dsb, semi)
        pltpu.async_copy(ei_hbm.at[1, pl.ds(off, CHUNK)], srb, semi)
        pltpu.async_copy(val_hbm.at[pl.ds(off, CHUNK)], vlb, semi)

    def wait_idx(k):
        dsb, srb, vlb, semi = sets[k]
        pltpu.make_async_copy(ei_hbm.at[0, pl.ds(0, CHUNK)], dsb, semi).wait()
        pltpu.make_async_copy(ei_hbm.at[1, pl.ds(0, CHUNK)], srb, semi).wait()
        pltpu.make_async_copy(val_hbm.at[pl.ds(0, CHUNK)], vlb, semi).wait()

    def start_gather(k, r):
        srb = sets[k][1]
        rowsb, semr = rbufs[r]
        pltpu.async_copy(emb_hbm.at[srb], rowsb, semr)

    def wait_gather(k, r):
        srb = sets[k][1]
        rowsb, semr = rbufs[r]
        pltpu.make_async_copy(emb_hbm.at[srb], rowsb, semr).wait()

    def scale_and_scatter(k, r):
        dsb, _, vlb, _ = sets[k]
        rowsb, _ = rbufs[r]

        def scale_group(g, carry2):
            vv = vlb[pl.ds(g * 16, 16)]
            for i in range(16):
                v = vv[i]
                e = g * 16 + i
                for j in range(D_FEAT // 16):
                    sl = pl.ds(j * 16, 16)
                    rowsb[e, sl] = rowsb[e, sl] * v
            return carry2

        lax.fori_loop(0, CHUNK // 16, scale_group, 0, unroll=2)
        semw = wsems[r]
        pltpu.async_copy(rowsb, acc_sh.at[dsb], semw, add=True)

    def scatter_wait(k, r):
        dsb = sets[k][0]
        rowsb, _ = rbufs[r]
        pltpu.make_async_copy(rowsb, acc_sh.at[dsb], wsems[r]).wait()

    for k in range(3):
        fire_idx(k, k)

    z = jnp.zeros((16,), jnp.float32)

    def zfill(i, carry):
        for j in range(D_FEAT // 16):
            zbuf_v[i, pl.ds(j * 16, 16)] = z
        return carry

    with jax.named_scope("acc_zero"):
        lax.fori_loop(0, ZROWS, zfill, 0)
        slab0 = pl.multiple_of(s * ROWS_PER_TILE, 8)
        for j in range(ROWS_PER_TILE // ZROWS):
            pltpu.async_copy(
                zbuf_v, acc_sh.at[pl.ds(slab0 + j * ZROWS, ZROWS)], semz)
        for j in range(ROWS_PER_TILE // ZROWS):
            pltpu.make_async_copy(
                zbuf_v, acc_sh.at[pl.ds(slab0 + j * ZROWS, ZROWS)], semz).wait()
        plsc.subcore_barrier()

    wait_idx(0)
    start_gather(0, 0)

    def body(i4, carry):
        ci0 = i4 * 4

        def step(koff, r, r_other):
            k = koff
            knext = (koff + 1) % 4
            kprev = (koff + 3) % 4

            if koff == 0:
                @pl.when(ci0 > 0)
                def _():
                    scatter_wait(kprev, r_other)
            else:
                scatter_wait(kprev, r_other)

            @pl.when(ci0 + koff + 3 < nck)
            def _():
                fire_idx(ci0 + koff + 3, kprev)

            if koff < 3:
                wait_idx(knext)
                start_gather(knext, r_other)
            else:
                @pl.when(ci0 + 4 < nck)
                def _():
                    wait_idx(0)
                    start_gather(0, r_other)

            wait_gather(k, r)
            scale_and_scatter(k, r)

        step(0, 0, 1)
        step(1, 1, 0)
        step(2, 0, 1)
        step(3, 1, 0)
        return carry

    with jax.named_scope("edge_loop"):
        lax.fori_loop(0, nck // 4, body, 0)
        scatter_wait(3, 1)
    plsc.subcore_barrier()

    with jax.named_scope("acc_drain"):
        pltpu.sync_copy(acc_sh.at[pl.ds(slab0, ROWS_PER_TILE)],
                        out_hbm.at[c, pl.ds(slab0, ROWS_PER_TILE)])


def _tc_add(a_ref, b_ref, o_ref):
    o_ref[...] = a_ref[...] + b_ref[...]


def kernel(edge_index, edge_values, embeds):
    ei = edge_index.astype(jnp.int32)
    val = edge_values.astype(jnp.float32)

    mesh = plsc.VectorSubcoreMesh(core_axis_name="c", subcore_axis_name="s")
    idx_set = [pltpu.VMEM((CHUNK,), jnp.int32),
               pltpu.VMEM((CHUNK,), jnp.int32),
               pltpu.VMEM((CHUNK,), jnp.float32)]
    partials = pl.kernel(
        _sc_spmm,
        mesh=mesh,
        out_type=jax.ShapeDtypeStruct((NUM_CORES, N_ROWS_PAD, D_FEAT), jnp.float32),
        scratch_types=[
            *idx_set, *idx_set, *idx_set, *idx_set,
            pltpu.VMEM((CHUNK, D_FEAT), jnp.float32),
            pltpu.VMEM((CHUNK, D_FEAT), jnp.float32),
            pltpu.VMEM((ZROWS, D_FEAT), jnp.float32),
            pltpu.VMEM_SHARED((N_ROWS_PAD, D_FEAT), jnp.float32),
            pltpu.SemaphoreType.DMA,
            pltpu.SemaphoreType.DMA,
            pltpu.SemaphoreType.DMA,
            pltpu.SemaphoreType.DMA,
            pltpu.SemaphoreType.DMA,
            pltpu.SemaphoreType.DMA,
            pltpu.SemaphoreType.DMA,
            pltpu.SemaphoreType.DMA,
            pltpu.SemaphoreType.DMA,
        ],
    )(ei, val, embeds)

    rows_blk = 1000
    out = pl.pallas_call(
        _tc_add,
        grid=(N_NODES // rows_blk,),
        in_specs=[
            pl.BlockSpec((rows_blk, D_FEAT), lambda i: (i, 0)),
            pl.BlockSpec((rows_blk, D_FEAT), lambda i: (i, 0)),
        ],
        out_specs=pl.BlockSpec((rows_blk, D_FEAT), lambda i: (i, 0)),
        out_shape=jax.ShapeDtypeStruct((N_NODES, D_FEAT), jnp.float32),
    )(partials[0], partials[1])
    return out


# speedup vs baseline: 1.0703x; 1.0703x over previous
import functools

import jax
import jax.numpy as jnp
from jax import lax
from jax.experimental import pallas as pl
from jax.experimental.pallas import tpu as pltpu
from jax.experimental.pallas import tpu_sc as plsc

N_NODES = 10000
N_EDGES = 320000
D_FEAT = 128

NUM_CORES = 2
NUM_SUBCORES = 16
NUM_WORKERS = NUM_CORES * NUM_SUBCORES
CHUNK = 80
NUM_CHUNKS = 128
EPW = NUM_CHUNKS * CHUNK
LAST_CHUNKS = (N_EDGES - (NUM_WORKERS - 1) * EPW) // CHUNK
N_ROWS_PAD = 10240
ROWS_PER_TILE = N_ROWS_PAD // NUM_SUBCORES
ZROWS = 32
NSETS = 8
NROWS = 4


def _sc_spmm(ei_hbm, val_hbm, emb_hbm, out_hbm, *refs):
    idx_bufs = refs[0:3 * NSETS]
    rows = refs[3 * NSETS:3 * NSETS + NROWS]
    zbuf_v = refs[3 * NSETS + NROWS]
    acc_sh = refs[3 * NSETS + NROWS + 1]
    sems = refs[3 * NSETS + NROWS + 2:]
    isems = sems[0:NSETS]
    gsems = sems[NSETS:NSETS + NROWS]
    wsems = sems[NSETS + NROWS:NSETS + 2 * NROWS]
    semz = sems[NSETS + 2 * NROWS]
    sets = tuple(
        (idx_bufs[3 * k], idx_bufs[3 * k + 1], idx_bufs[3 * k + 2], isems[k])
        for k in range(NSETS))

    c = lax.axis_index("c")
    s = lax.axis_index("s")
    wid = c * NUM_SUBCORES + s
    ebase = wid * EPW
    nck = jnp.where(wid == NUM_WORKERS - 1, LAST_CHUNKS, NUM_CHUNKS)

    def fire_idx(ci, k):
        dsb, srb, vlb, semi = sets[k]
        off = pl.multiple_of(ebase + ci * CHUNK, 8)
        pltpu.async_copy(ei_hbm.at[pl.ds(off, CHUNK)], dsb, semi)
        pltpu.async_copy(ei_hbm.at[pl.ds(off + N_EDGES, CHUNK)], srb, semi)
        pltpu.async_copy(val_hbm.at[pl.ds(off, CHUNK)], vlb, semi)

    def wait_idx(k):
        dsb, srb, vlb, semi = sets[k]
        pltpu.make_async_copy(ei_hbm.at[pl.ds(0, CHUNK)], dsb, semi).wait()
        pltpu.make_async_copy(ei_hbm.at[pl.ds(0, CHUNK)], srb, semi).wait()
        pltpu.make_async_copy(val_hbm.at[pl.ds(0, CHUNK)], vlb, semi).wait()

    def start_gather(k, r):
        pltpu.async_copy(emb_hbm.at[sets[k][1]], rows[r], gsems[r])

    def wait_gather(k, r):
        pltpu.make_async_copy(emb_hbm.at[sets[k][1]], rows[r], gsems[r]).wait()

    def scale_and_scatter(k, r):
        dsb, _, vlb, _ = sets[k]
        rowsb = rows[r]

        def scale_group(g, carry2):
            vv = vlb[pl.ds(g * 16, 16)]
            for i in range(16):
                v = vv[i]
                e = g * 16 + i
                for j in range(D_FEAT // 16):
                    sl = pl.ds(j * 16, 16)
                    rowsb[e, sl] = rowsb[e, sl] * v
            return carry2

        lax.fori_loop(0, CHUNK // 16, scale_group, 0)
        pltpu.async_copy(rowsb, acc_sh.at[dsb], wsems[r], add=True)

    def scatter_wait(k, r):
        pltpu.make_async_copy(rows[r], acc_sh.at[sets[k][0]], wsems[r]).wait()

    for k in range(5):
        fire_idx(k, k)

    z = jnp.zeros((16,), jnp.float32)

    def zfill(i, carry):
        for j in range(D_FEAT // 16):
            zbuf_v[i, pl.ds(j * 16, 16)] = z
        return carry

    with jax.named_scope("acc_zero"):
        lax.fori_loop(0, ZROWS, zfill, 0)
        slab0 = pl.multiple_of(s * ROWS_PER_TILE, 8)
        for j in range(ROWS_PER_TILE // ZROWS):
            pltpu.async_copy(
                zbuf_v, acc_sh.at[pl.ds(slab0 + j * ZROWS, ZROWS)], semz)
        for j in range(ROWS_PER_TILE // ZROWS):
            pltpu.make_async_copy(
                zbuf_v, acc_sh.at[pl.ds(slab0 + j * ZROWS, ZROWS)], semz).wait()
        plsc.subcore_barrier()

    wait_idx(0)
    start_gather(0, 0)
    wait_idx(1)
    start_gather(1, 1)

    def body(i8, carry):
        ci0 = i8 * 8

        def step(koff):
            k = koff
            r = koff % 4
            cc = ci0 + koff

            kw = (koff + 6) % 8
            rw = (koff + 2) % 4
            if koff < 2:
                @pl.when(ci0 > 0)
                def _():
                    scatter_wait(kw, rw)
            else:
                scatter_wait(kw, rw)

            @pl.when(cc + 5 < nck)
            def _():
                fire_idx(cc + 5, (koff + 5) % 8)

            @pl.when(cc + 2 < nck)
            def _():
                wait_idx((koff + 2) % 8)
                start_gather((koff + 2) % 8, rw)

            wait_gather(k, r)
            scale_and_scatter(k, r)

        for koff in range(8):
            step(koff)
        return carry

    with jax.named_scope("edge_loop"):
        lax.fori_loop(0, nck // 8, body, 0)
        scatter_wait(6, 2)
        scatter_wait(7, 3)
    plsc.subcore_barrier()

    with jax.named_scope("acc_drain"):
        pltpu.sync_copy(acc_sh.at[pl.ds(slab0, ROWS_PER_TILE)],
                        out_hbm.at[c, pl.ds(slab0, ROWS_PER_TILE)])


def _tc_add(a_ref, b_ref, o_ref):
    o_ref[...] = a_ref[...] + b_ref[...]


def kernel(edge_index, edge_values, embeds):
    ei = edge_index.astype(jnp.int32).reshape(-1)
    val = edge_values.astype(jnp.float32)

    mesh = plsc.VectorSubcoreMesh(core_axis_name="c", subcore_axis_name="s")
    idx_set = [pltpu.VMEM((CHUNK,), jnp.int32),
               pltpu.VMEM((CHUNK,), jnp.int32),
               pltpu.VMEM((CHUNK,), jnp.float32)]
    partials = pl.kernel(
        _sc_spmm,
        mesh=mesh,
        out_type=jax.ShapeDtypeStruct((NUM_CORES, N_ROWS_PAD, D_FEAT), jnp.float32),
        scratch_types=[
            *(idx_set * NSETS),
            *([pltpu.VMEM((CHUNK, D_FEAT), jnp.float32)] * NROWS),
            pltpu.VMEM((ZROWS, D_FEAT), jnp.float32),
            pltpu.VMEM_SHARED((N_ROWS_PAD, D_FEAT), jnp.float32),
            *([pltpu.SemaphoreType.DMA] * (NSETS + 2 * NROWS + 1)),
        ],
    )(ei, val, embeds)

    rows_blk = 1000
    out = pl.pallas_call(
        _tc_add,
        grid=(N_NODES // rows_blk,),
        in_specs=[
            pl.BlockSpec((rows_blk, D_FEAT), lambda i: (i, 0)),
            pl.BlockSpec((rows_blk, D_FEAT), lambda i: (i, 0)),
        ],
        out_specs=pl.BlockSpec((rows_blk, D_FEAT), lambda i: (i, 0)),
        out_shape=jax.ShapeDtypeStruct((N_NODES, D_FEAT), jnp.float32),
    )(partials[0], partials[1])
    return out


# speedup vs baseline: 1.0882x; 1.0168x over previous
import functools

import jax
import jax.numpy as jnp
from jax import lax
from jax.experimental import pallas as pl
from jax.experimental.pallas import tpu as pltpu
from jax.experimental.pallas import tpu_sc as plsc

N_NODES = 10000
N_EDGES = 320000
D_FEAT = 128

NUM_CORES = 2
NUM_SUBCORES = 16
NUM_WORKERS = NUM_CORES * NUM_SUBCORES
CHUNK = 80
NUM_CHUNKS = 128
EPW = NUM_CHUNKS * CHUNK
LAST_CHUNKS = (N_EDGES - (NUM_WORKERS - 1) * EPW) // CHUNK
N_ROWS_PAD = 10240
ROWS_PER_TILE = N_ROWS_PAD // NUM_SUBCORES
ZROWS = 32
NSETS = 8
NROWS = 4


def _sc_spmm(ei_hbm, val_hbm, emb_hbm, out_hbm, *refs):
    idx_bufs = refs[0:3 * NSETS]
    rows = refs[3 * NSETS:3 * NSETS + NROWS]
    zbuf_v = refs[3 * NSETS + NROWS]
    acc_sh = refs[3 * NSETS + NROWS + 1]
    sems = refs[3 * NSETS + NROWS + 2:]
    isems = sems[0:NSETS]
    gsems = sems[NSETS:NSETS + NROWS]
    wsems = sems[NSETS + NROWS:NSETS + 2 * NROWS]
    semz = sems[NSETS + 2 * NROWS]
    sets = tuple(
        (idx_bufs[3 * k], idx_bufs[3 * k + 1], idx_bufs[3 * k + 2], isems[k])
        for k in range(NSETS))

    c = lax.axis_index("c")
    s = lax.axis_index("s")
    wid = c * NUM_SUBCORES + s
    ebase = wid * EPW
    nck = jnp.where(wid == NUM_WORKERS - 1, LAST_CHUNKS, NUM_CHUNKS)

    def fire_idx(ci, k):
        dsb, srb, vlb, semi = sets[k]
        off = pl.multiple_of(ebase + ci * CHUNK, 8)
        pltpu.async_copy(ei_hbm.at[pl.ds(off, CHUNK)], dsb, semi)
        pltpu.async_copy(ei_hbm.at[pl.ds(off + N_EDGES, CHUNK)], srb, semi)
        pltpu.async_copy(val_hbm.at[pl.ds(off, CHUNK)], vlb, semi)

    def wait_idx(k):
        dsb, srb, vlb, semi = sets[k]
        pltpu.make_async_copy(ei_hbm.at[pl.ds(0, CHUNK)], dsb, semi).wait()
        pltpu.make_async_copy(ei_hbm.at[pl.ds(0, CHUNK)], srb, semi).wait()
        pltpu.make_async_copy(val_hbm.at[pl.ds(0, CHUNK)], vlb, semi).wait()

    def start_gather(k, r):
        pltpu.async_copy(emb_hbm.at[sets[k][1]], rows[r], gsems[r])

    def wait_gather(k, r):
        pltpu.make_async_copy(emb_hbm.at[sets[k][1]], rows[r], gsems[r]).wait()

    def scale_and_scatter(k, r):
        dsb, _, vlb, _ = sets[k]
        rowsb = rows[r]

        def scale_group(g, carry2):
            vv = vlb[pl.ds(g * 16, 16)]
            for i in range(16):
                v = vv[i]
                e = g * 16 + i
                for j in range(D_FEAT // 16):
                    sl = pl.ds(j * 16, 16)
                    rowsb[e, sl] = rowsb[e, sl] * v
            return carry2

        lax.fori_loop(0, CHUNK // 16, scale_group, 0)
        pltpu.async_copy(rowsb, acc_sh.at[dsb], wsems[r], add=True)

    def scatter_wait(k, r):
        pltpu.make_async_copy(rows[r], acc_sh.at[sets[k][0]], wsems[r]).wait()

    for k in range(5):
        fire_idx(k, k)

    z = jnp.zeros((16,), jnp.float32)

    def zfill(i, carry):
        for j in range(D_FEAT // 16):
            zbuf_v[i, pl.ds(j * 16, 16)] = z
        return carry

    with jax.named_scope("acc_zero"):
        lax.fori_loop(0, ZROWS, zfill, 0)
        slab0 = pl.multiple_of(s * ROWS_PER_TILE, 8)
        for j in range(ROWS_PER_TILE // ZROWS):
            pltpu.async_copy(
                zbuf_v, acc_sh.at[pl.ds(slab0 + j * ZROWS, ZROWS)], semz)
        for j in range(ROWS_PER_TILE // ZROWS):
            pltpu.make_async_copy(
                zbuf_v, acc_sh.at[pl.ds(slab0 + j * ZROWS, ZROWS)], semz).wait()
        plsc.subcore_barrier()

    wait_idx(0)
    start_gather(0, 0)
    wait_idx(1)
    start_gather(1, 1)

    def body(i8, carry):
        ci0 = i8 * 8

        def step(koff):
            k = koff
            r = koff % 4
            cc = ci0 + koff

            kw = (koff + 6) % 8
            rw = (koff + 2) % 4
            if koff < 2:
                @pl.when(ci0 > 0)
                def _():
                    scatter_wait(kw, rw)
            else:
                scatter_wait(kw, rw)

            @pl.when(cc + 5 < nck)
            def _():
                fire_idx(cc + 5, (koff + 5) % 8)

            @pl.when(cc + 2 < nck)
            def _():
                wait_idx((koff + 2) % 8)
                start_gather((koff + 2) % 8, rw)

            wait_gather(k, r)
            scale_and_scatter(k, r)

        for koff in range(8):
            step(koff)
        return carry

    with jax.named_scope("edge_loop"):
        lax.fori_loop(0, nck // 8, body, 0)
        scatter_wait(6, 2)
        scatter_wait(7, 3)
    plsc.subcore_barrier()

    with jax.named_scope("acc_drain"):
        pltpu.sync_copy(acc_sh.at[pl.ds(slab0, ROWS_PER_TILE)],
                        out_hbm.at[c, pl.ds(slab0, ROWS_PER_TILE)])


def _tc_add(a_ref, b_ref, o_ref):
    o_ref[...] = a_ref[...] + b_ref[...]


def kernel(edge_index, edge_values, embeds):
    ei = edge_index.astype(jnp.int32).reshape(-1)
    val = edge_values.astype(jnp.float32)

    mesh = plsc.VectorSubcoreMesh(core_axis_name="c", subcore_axis_name="s")
    idx_set = [pltpu.VMEM((CHUNK,), jnp.int32),
               pltpu.VMEM((CHUNK,), jnp.int32),
               pltpu.VMEM((CHUNK,), jnp.float32)]
    partials = pl.kernel(
        _sc_spmm,
        mesh=mesh,
        out_type=jax.ShapeDtypeStruct((NUM_CORES, N_ROWS_PAD, D_FEAT), jnp.float32),
        scratch_types=[
            *(idx_set * NSETS),
            *([pltpu.VMEM((CHUNK, D_FEAT), jnp.float32)] * NROWS),
            pltpu.VMEM((ZROWS, D_FEAT), jnp.float32),
            pltpu.VMEM_SHARED((N_ROWS_PAD, D_FEAT), jnp.float32),
            *([pltpu.SemaphoreType.DMA] * (NSETS + 2 * NROWS + 1)),
        ],
    )(ei, val, embeds)

    rows_blk = 2000
    out = pl.pallas_call(
        _tc_add,
        grid=(N_NODES // rows_blk,),
        in_specs=[
            pl.BlockSpec((rows_blk, D_FEAT), lambda i: (i, 0)),
            pl.BlockSpec((rows_blk, D_FEAT), lambda i: (i, 0)),
        ],
        out_specs=pl.BlockSpec((rows_blk, D_FEAT), lambda i: (i, 0)),
        out_shape=jax.ShapeDtypeStruct((N_NODES, D_FEAT), jnp.float32),
    )(partials[0], partials[1])
    return out
